# baseline (speedup 1.0000x reference)
import functools

import jax
import jax.numpy as jnp
from jax import lax
from jax.experimental import pallas as pl
from jax.experimental.pallas import tpu as pltpu
from jax.experimental.pallas import tpu_sc as plsc

NC = 2
NS = 16
L = 16
RING = 4
GRP = 1
CHUNKS = (384, 256)
PASSES = 2


@functools.lru_cache(maxsize=None)
def _build(N, D, K, chunks, iw, aw):
    g = K + 1
    glen = (GRP - 1) * iw + g
    dc = D // L
    cmax = max(chunks) // PASSES
    b_pad = NS * sum(chunks)
    mesh = plsc.VectorSubcoreMesh(core_axis_name="c", subcore_axis_name="s")

    @functools.partial(
        pl.kernel,
        mesh=mesh,
        out_type=jax.ShapeDtypeStruct((b_pad * D,), jnp.float32),
        compiler_params=pltpu.CompilerParams(needs_layout_passes=False),
        scratch_types=[
            pltpu.VMEM((cmax * iw,), jnp.int32),
            pltpu.VMEM((cmax * K,), jnp.float32),
            pltpu.VMEM((cmax * aw,), jnp.float32),
            pltpu.VMEM((cmax * D,), jnp.float32),
        ] + [pltpu.VMEM((glen, D), jnp.float32) for _ in range(RING)]
          + [pltpu.SemaphoreType.DMA for _ in range(2 * RING)],
    )
    def aggregate(idx_hbm, wt_hbm, table_hbm, out_hbm,
                  idx_v, wt_v, a_v, out_v, *bufs_sems):
        bufs = bufs_sems[:RING]
        sems = [bufs_sems[RING + 2 * r: RING + 2 * r + 2] for r in range(RING)]
        cid = lax.axis_index("c")
        sid = lax.axis_index("s")
        lane_off = lax.iota(jnp.int32, L) * aw

        def work(base, chunk):
            pltpu.sync_copy(idx_hbm.at[pl.ds(base * iw, chunk * iw)], idx_v.at[pl.ds(0, chunk * iw)])
            pltpu.sync_copy(wt_hbm.at[pl.ds(base * K, chunk * K)], wt_v.at[pl.ds(0, chunk * K)])

            def norm_body(t, carry):
                wb = t * K * L
                wt = [wt_v[pl.ds(wb + j * L, L)] for j in range(K)]
                tot = wt[0]
                for j in range(1, K):
                    tot = tot + wt[j]
                safe = jnp.where(tot == 0.0, jnp.float32(1.0), tot)
                inv_total = jnp.float32(1.0) / safe
                rs = tot * inv_total + jnp.float32(1.0)
                inv_rs = jnp.float32(1.0) / rs
                s = inv_total * inv_rs
                ab = t * L * aw
                for j in range(K):
                    plsc.store_scatter(a_v, [lane_off + (ab + j)], wt[j] * s)
                plsc.store_scatter(a_v, [lane_off + (ab + K)], inv_rs)
                return carry

            lax.fori_loop(0, chunk // L, norm_body, 0)

            npairs = chunk // GRP

            sa = glen // 2 // 8 * 8
            sb = glen - sa

            def gather_start(p, buf, sem):
                pltpu.make_async_copy(
                    table_hbm.at[idx_v.at[pl.ds(p * GRP * iw, sa)]],
                    buf.at[pl.ds(0, sa)], sem[0]).start()
                pltpu.make_async_copy(
                    table_hbm.at[idx_v.at[pl.ds(p * GRP * iw + sa, sb)]],
                    buf.at[pl.ds(sa, sb)], sem[1]).start()

            def gather_wait(p, buf, sem):
                pltpu.make_async_copy(
                    table_hbm.at[idx_v.at[pl.ds(p * GRP * iw, sa)]],
                    buf.at[pl.ds(0, sa)], sem[0]).wait()
                pltpu.make_async_copy(
                    table_hbm.at[idx_v.at[pl.ds(p * GRP * iw + sa, sb)]],
                    buf.at[pl.ds(sa, sb)], sem[1]).wait()

            def compute(i, rows, roff):
                ab = i * aw
                av = [a_v[pl.ds(ab + c * L, L)] for c in range(K // L + 1)]
                als = [av[c][l] for c in range(K // L) for l in range(L)]
                a_self = av[K // L][0]
                acc = [rows[roff + K, pl.ds(c * L, L)] * a_self for c in range(dc)]
                for j in range(K):
                    for c in range(dc):
                        acc[c] = acc[c] + rows[roff + j, pl.ds(c * L, L)] * als[j]
                for c in range(dc):
                    out_v[pl.ds(i * D + c * L, L)] = acc[c]

            for r in range(RING):
                gather_start(r, bufs[r], sems[r])

            def body(t, carry):
                p0 = RING * t
                for r in range(RING):
                    gather_wait(p0 + r, bufs[r], sems[r])
                    for q in range(GRP):
                        compute((p0 + r) * GRP + q, bufs[r], q * iw)
                    gather_start(jnp.minimum(p0 + r + RING, npairs - 1),
                                 bufs[r], sems[r])
                return carry

            lax.fori_loop(0, npairs // RING, body, 0)
            for r in range(RING):
                gather_wait(npairs - 1, bufs[r], sems[r])
            pltpu.sync_copy(out_v.at[pl.ds(0, chunk * D)], out_hbm.at[pl.ds(base * D, chunk * D)])

        @pl.when(cid == 0)
        def _core0():
            sub = chunks[0] // PASSES

            def pass_body(h, carry):
                work(sid * chunks[0] + h * sub, sub)
                return carry

            lax.fori_loop(0, PASSES, pass_body, 0)

        @pl.when(cid == 1)
        def _core1():
            sub = chunks[1] // PASSES

            def pass_body(h, carry):
                work(NS * chunks[0] + sid * chunks[1] + h * sub, sub)
                return carry

            lax.fori_loop(0, PASSES, pass_body, 0)

    return aggregate


def kernel(nodes, neigh_idx, neigh_w, feat_table):
    B, K = neigh_idx.shape
    N, D = feat_table.shape
    iw = K + 1
    while (GRP * iw) % 8:
        iw += 1
    aw = -(-(K + 1) // L) * L
    b_pad = NS * sum(CHUNKS)
    assert b_pad >= B
    idx = jnp.concatenate(
        [neigh_idx.astype(jnp.int32),
         nodes.astype(jnp.int32)[:, None],
         jnp.zeros((B, iw - K - 1), jnp.int32)], axis=1)
    idx = jnp.pad(idx, ((0, b_pad - B), (0, 0))).reshape(-1)
    w = jnp.pad(neigh_w.astype(jnp.float32), ((0, b_pad - B), (0, 0)))
    wt = w.reshape(b_pad // L, L, K).transpose(0, 2, 1).reshape(-1)
    out = _build(N, D, K, CHUNKS, iw, aw)(idx, wt, feat_table)
    return out.reshape(b_pad, D)[:B]


# speedup vs baseline: 1.0237x; 1.0237x over previous
import functools

import jax
import jax.numpy as jnp
from jax import lax
from jax.experimental import pallas as pl
from jax.experimental.pallas import tpu as pltpu
from jax.experimental.pallas import tpu_sc as plsc

NC = 2
NS = 16
L = 16
RING = 4
GRP = 1
CHUNKS = (512, 128)
PASSES = 2


@functools.lru_cache(maxsize=None)
def _build(N, D, K, chunks, iw, aw):
    g = K + 1
    glen = (GRP - 1) * iw + g
    dc = D // L
    cmax = max(chunks) // PASSES
    b_pad = NS * sum(chunks)
    mesh = plsc.VectorSubcoreMesh(core_axis_name="c", subcore_axis_name="s")

    @functools.partial(
        pl.kernel,
        mesh=mesh,
        out_type=jax.ShapeDtypeStruct((b_pad * D,), jnp.float32),
        compiler_params=pltpu.CompilerParams(needs_layout_passes=False),
        scratch_types=[
            pltpu.VMEM((cmax * iw,), jnp.int32),
            pltpu.VMEM((cmax * K,), jnp.float32),
            pltpu.VMEM((cmax * aw,), jnp.float32),
            pltpu.VMEM((cmax * D,), jnp.float32),
        ] + [pltpu.VMEM((glen, D), jnp.float32) for _ in range(RING)]
          + [pltpu.SemaphoreType.DMA for _ in range(2 * RING)],
    )
    def aggregate(idx_hbm, wt_hbm, table_hbm, out_hbm,
                  idx_v, wt_v, a_v, out_v, *bufs_sems):
        bufs = bufs_sems[:RING]
        sems = [bufs_sems[RING + 2 * r: RING + 2 * r + 2] for r in range(RING)]
        cid = lax.axis_index("c")
        sid = lax.axis_index("s")
        lane_off = lax.iota(jnp.int32, L) * aw

        def work(base, chunk):
            pltpu.sync_copy(idx_hbm.at[pl.ds(base * iw, chunk * iw)], idx_v.at[pl.ds(0, chunk * iw)])
            pltpu.sync_copy(wt_hbm.at[pl.ds(base * K, chunk * K)], wt_v.at[pl.ds(0, chunk * K)])

            def norm_body(t, carry):
                wb = t * K * L
                wt = [wt_v[pl.ds(wb + j * L, L)] for j in range(K)]
                tot = wt[0]
                for j in range(1, K):
                    tot = tot + wt[j]
                safe = jnp.where(tot == 0.0, jnp.float32(1.0), tot)
                inv_total = jnp.float32(1.0) / safe
                rs = tot * inv_total + jnp.float32(1.0)
                inv_rs = jnp.float32(1.0) / rs
                s = inv_total * inv_rs
                ab = t * L * aw
                for j in range(K):
                    plsc.store_scatter(a_v, [lane_off + (ab + j)], wt[j] * s)
                plsc.store_scatter(a_v, [lane_off + (ab + K)], inv_rs)
                return carry

            lax.fori_loop(0, chunk // L, norm_body, 0)

            npairs = chunk // GRP

            sa = glen // 2 // 8 * 8
            sb = glen - sa

            def gather_start(p, buf, sem):
                pltpu.make_async_copy(
                    table_hbm.at[idx_v.at[pl.ds(p * GRP * iw, sa)]],
                    buf.at[pl.ds(0, sa)], sem[0]).start()
                pltpu.make_async_copy(
                    table_hbm.at[idx_v.at[pl.ds(p * GRP * iw + sa, sb)]],
                    buf.at[pl.ds(sa, sb)], sem[1]).start()

            def gather_wait(p, buf, sem):
                pltpu.make_async_copy(
                    table_hbm.at[idx_v.at[pl.ds(p * GRP * iw, sa)]],
                    buf.at[pl.ds(0, sa)], sem[0]).wait()
                pltpu.make_async_copy(
                    table_hbm.at[idx_v.at[pl.ds(p * GRP * iw + sa, sb)]],
                    buf.at[pl.ds(sa, sb)], sem[1]).wait()

            def compute(i, rows, roff):
                ab = i * aw
                av = [a_v[pl.ds(ab + c * L, L)] for c in range(K // L + 1)]
                als = [av[c][l] for c in range(K // L) for l in range(L)]
                a_self = av[K // L][0]
                acc = [rows[roff + K, pl.ds(c * L, L)] * a_self for c in range(dc)]
                for j in range(K):
                    for c in range(dc):
                        acc[c] = acc[c] + rows[roff + j, pl.ds(c * L, L)] * als[j]
                for c in range(dc):
                    out_v[pl.ds(i * D + c * L, L)] = acc[c]

            for r in range(RING):
                gather_start(r, bufs[r], sems[r])

            def body(t, carry):
                p0 = RING * t
                for r in range(RING):
                    gather_wait(p0 + r, bufs[r], sems[r])
                    for q in range(GRP):
                        compute((p0 + r) * GRP + q, bufs[r], q * iw)
                    gather_start(jnp.minimum(p0 + r + RING, npairs - 1),
                                 bufs[r], sems[r])
                return carry

            lax.fori_loop(0, npairs // RING, body, 0)
            for r in range(RING):
                gather_wait(npairs - 1, bufs[r], sems[r])
            pltpu.sync_copy(out_v.at[pl.ds(0, chunk * D)], out_hbm.at[pl.ds(base * D, chunk * D)])

        @pl.when(cid == 0)
        def _core0():
            sub = chunks[0] // PASSES

            def pass_body(h, carry):
                work(sid * chunks[0] + h * sub, sub)
                return carry

            lax.fori_loop(0, PASSES, pass_body, 0)

        @pl.when(cid == 1)
        def _core1():
            sub = chunks[1] // PASSES

            def pass_body(h, carry):
                work(NS * chunks[0] + sid * chunks[1] + h * sub, sub)
                return carry

            lax.fori_loop(0, PASSES, pass_body, 0)

    return aggregate


def kernel(nodes, neigh_idx, neigh_w, feat_table):
    B, K = neigh_idx.shape
    N, D = feat_table.shape
    iw = K + 1
    while (GRP * iw) % 8:
        iw += 1
    aw = -(-(K + 1) // L) * L
    b_pad = NS * sum(CHUNKS)
    assert b_pad >= B
    idx = jnp.concatenate(
        [neigh_idx.astype(jnp.int32),
         nodes.astype(jnp.int32)[:, None],
         jnp.zeros((B, iw - K - 1), jnp.int32)], axis=1)
    idx = jnp.pad(idx, ((0, b_pad - B), (0, 0))).reshape(-1)
    w = jnp.pad(neigh_w.astype(jnp.float32), ((0, b_pad - B), (0, 0)))
    wt = w.reshape(b_pad // L, L, K).transpose(0, 2, 1).reshape(-1)
    out = _build(N, D, K, CHUNKS, iw, aw)(idx, wt, feat_table)
    return out.reshape(b_pad, D)[:B]
